# Initial kernel scaffold; baseline (speedup 1.0000x reference)
import jax
import jax.numpy as jnp
from jax.experimental import pallas as pl


def kernel(treatment_node_features, treatment_edges, edge_types, batch_assignments, W1, b1, W2, b2):
    raise NotImplementedError("write your pallas kernel here")


# speedup vs baseline: 4.1757x; 4.1757x over previous
import functools

import jax
import jax.numpy as jnp
from jax import lax
from jax.experimental import pallas as pl
from jax.experimental.pallas import tpu as pltpu
from jax.experimental.pallas import tpu_sc as plsc

N = 10000
D = 128
E = 320000
G = 128

NC = 2
NS = 16
NW = NC * NS
CHUNK = 128
CHUNKS_PER_W = -(-E // (NW * CHUNK))
EPW = CHUNKS_PER_W * CHUNK
E_PAD = EPW * NW
STRIPE = 640
N_PAD = STRIPE * NS

R = 2000


def _sc_segsum_body(x_hbm, src_hbm, dst_hbm, out_hbm,
                    acc, src_v, dst_v, rows_v, sem):
    c = lax.axis_index("c")
    s = lax.axis_index("s")
    wid = s * NC + c

    def zero_row(i, _):
        for j in range(D // 16):
            rows_v[i, pl.ds(j * 16, 16)] = jnp.zeros((16,), jnp.float32)
        return 0
    lax.fori_loop(0, CHUNK, zero_row, 0)
    for j in range(STRIPE // CHUNK):
        pltpu.sync_copy(rows_v, acc.at[pl.ds(s * STRIPE + j * CHUNK, CHUNK), :])
    plsc.subcore_barrier()

    def body(k, _):
        base = wid * EPW + k * CHUNK
        pltpu.sync_copy(src_hbm.at[pl.ds(base, CHUNK)], src_v)
        pltpu.sync_copy(dst_hbm.at[pl.ds(base, CHUNK)], dst_v)
        pltpu.async_copy(x_hbm.at[src_v], rows_v, sem).wait()
        pltpu.sync_copy(rows_v, acc.at[dst_v], add=True)
        return 0
    lax.fori_loop(0, CHUNKS_PER_W, body, 0)
    plsc.subcore_barrier()

    pltpu.sync_copy(acc.at[pl.ds(s * STRIPE, STRIPE), :],
                    out_hbm.at[c, pl.ds(s * STRIPE, STRIPE), :])


@functools.partial(jax.jit, static_argnames=())
def _sc_segsum(x, src_p, dst_p):
    mesh = plsc.VectorSubcoreMesh(core_axis_name="c", subcore_axis_name="s")
    return pl.kernel(
        _sc_segsum_body,
        out_type=jax.ShapeDtypeStruct((NC, N_PAD, D), jnp.float32),
        mesh=mesh,
        scratch_types=[
            pltpu.VMEM_SHARED((N_PAD, D), jnp.float32),
            pltpu.VMEM((CHUNK,), jnp.int32),
            pltpu.VMEM((CHUNK,), jnp.int32),
            pltpu.VMEM((CHUNK, D), jnp.float32),
            pltpu.SemaphoreType.DMA,
        ],
    )(x, src_p, dst_p)


def _tc_layer_body(x_ref, p_ref, w_ref, b_ref, o_ref):
    acc = x_ref[...] + p_ref[0] + p_ref[1]
    h = jnp.dot(acc, w_ref[...], preferred_element_type=jnp.float32)
    o_ref[...] = jnp.maximum(h + b_ref[...], 0.0)


def _tc_layer(x, p, w, b):
    return pl.pallas_call(
        _tc_layer_body,
        grid=(N // R,),
        in_specs=[
            pl.BlockSpec((R, D), lambda i: (i, 0)),
            pl.BlockSpec((NC, R, D), lambda i: (0, i, 0)),
            pl.BlockSpec((D, D), lambda i: (0, 0)),
            pl.BlockSpec((1, D), lambda i: (0, 0)),
        ],
        out_specs=pl.BlockSpec((R, D), lambda i: (i, 0)),
        out_shape=jax.ShapeDtypeStruct((N, D), jnp.float32),
    )(x, p, w, b.reshape(1, D))


def _tc_layer_pool_body(h_ref, p_ref, w_ref, b_ref, bat_ref, o_ref):
    i = pl.program_id(0)
    acc = h_ref[...] + p_ref[0] + p_ref[1]
    h2 = jnp.maximum(
        jnp.dot(acc, w_ref[...], preferred_element_type=jnp.float32)
        + b_ref[...], 0.0)
    onehot = (lax.broadcasted_iota(jnp.int32, (G, 1), 0)
              == bat_ref[0]).astype(jnp.float32)
    part = jnp.dot(onehot, h2, preferred_element_type=jnp.float32)

    @pl.when(i == 0)
    def _():
        o_ref[...] = jnp.zeros_like(o_ref)
    o_ref[...] += part


def _tc_layer_pool(h, p, w, b, batch_row):
    return pl.pallas_call(
        _tc_layer_pool_body,
        grid=(N // R,),
        in_specs=[
            pl.BlockSpec((R, D), lambda i: (i, 0)),
            pl.BlockSpec((NC, R, D), lambda i: (0, i, 0)),
            pl.BlockSpec((D, D), lambda i: (0, 0)),
            pl.BlockSpec((1, D), lambda i: (0, 0)),
            pl.BlockSpec((1, 1, R), lambda i: (i, 0, 0)),
        ],
        out_specs=pl.BlockSpec((G, D), lambda i: (0, 0)),
        out_shape=jax.ShapeDtypeStruct((G, D), jnp.float32),
    )(h, p, w, b.reshape(1, D), batch_row)


def kernel(treatment_node_features, treatment_edges, edge_types,
           batch_assignments, W1, b1, W2, b2):
    del edge_types
    x = treatment_node_features
    src = treatment_edges[0].astype(jnp.int32)
    dst = treatment_edges[1].astype(jnp.int32)
    batch_row = batch_assignments.astype(jnp.int32).reshape(N // R, 1, R)

    pad = E_PAD - E
    src_p = jnp.concatenate([src, jnp.zeros((pad,), jnp.int32)])
    dst_p = jnp.concatenate([dst, jnp.full((pad,), N, jnp.int32)])

    p1 = _sc_segsum(x, src_p, dst_p)
    h = _tc_layer(x, p1, W1, b1)
    p2 = _sc_segsum(h, src_p, dst_p)
    return _tc_layer_pool(h, p2, W2, b2, batch_row)
